# baseline (speedup 1.0000x reference)
import functools

import jax
import jax.numpy as jnp
from jax import lax
from jax.experimental import pallas as pl
from jax.experimental.pallas import tpu as pltpu
from jax.experimental.pallas import tpu_sc as plsc

B = 16384
D = 64
N_USERS = 1000000
N_ANIME = 100000
NC = 2
NS = 16
NW = NC * NS
BPW = B // NW
CHUNK = 128
N_CHUNKS = BPW // CHUNK
LANES = 16
GROUPS = CHUNK // LANES
D_UNROLL = 4


def _mf_body(user_hbm, anime_hbm, uf_hbm, af_hbm, out_hbm,
             uidx, aidx, uwidx, awidx, uvals, avals, outv, sems):
    wid = lax.axis_index("s") * NC + lax.axis_index("c")
    base = pl.multiple_of(wid * BPW, BPW)

    for k in range(N_CHUNKS):
        pltpu.sync_copy(user_hbm.at[pl.ds(base + k * CHUNK, CHUNK)],
                        uidx.at[k])
        pltpu.sync_copy(anime_hbm.at[pl.ds(base + k * CHUNK, CHUNK)],
                        aidx.at[k])

    def build_indices(k):
        buf = k % 2

        def per_d(d, carry):
            for g in range(GROUPS):
                sl = pl.ds(g * LANES, LANES)
                uwidx[buf, d, sl] = uidx[k, sl] + d * N_USERS
                awidx[buf, d, sl] = aidx[k, sl] + d * N_ANIME
            return carry
        lax.fori_loop(0, D, per_d, 0)

    def fire(k):
        buf = k % 2

        def per_d(d, carry):
            pltpu.async_copy(uf_hbm.at[uwidx.at[buf, d]], uvals.at[buf, d],
                             sems.at[buf, 0])
            pltpu.async_copy(af_hbm.at[awidx.at[buf, d]], avals.at[buf, d],
                             sems.at[buf, 1])
            return carry
        lax.fori_loop(0, D, per_d, 0)

    def drain(k):
        buf = k % 2

        def per_d(d, carry):
            pltpu.make_async_copy(uf_hbm.at[uwidx.at[buf, d]],
                                  uvals.at[buf, d], sems.at[buf, 0]).wait()
            pltpu.make_async_copy(af_hbm.at[awidx.at[buf, d]],
                                  avals.at[buf, d], sems.at[buf, 1]).wait()
            return carry
        lax.fori_loop(0, D, per_d, 0)

    build_indices(0)
    fire(0)

    for k in range(N_CHUNKS):
        buf = k % 2
        if k + 1 < N_CHUNKS:
            build_indices(k + 1)
        drain(k)
        if k + 1 < N_CHUNKS:
            fire(k + 1)

        def group(g, carry, buf=buf):
            gbase = pl.multiple_of(g * LANES, LANES)
            sl = pl.ds(gbase, LANES)

            def per_d(d, acc, sl=sl):
                dd = pl.multiple_of(d * D_UNROLL, D_UNROLL)
                for i in range(D_UNROLL):
                    acc = acc + uvals[buf, dd + i, sl] * avals[buf, dd + i, sl]
                return acc

            acc = lax.fori_loop(0, D // D_UNROLL, per_d,
                                jnp.zeros((LANES,), jnp.float32))
            outv[pl.ds(pl.multiple_of(k * CHUNK, CHUNK) + gbase, LANES)] = acc
            return carry

        lax.fori_loop(0, GROUPS, group, 0)

    pltpu.sync_copy(outv, out_hbm.at[pl.ds(base, BPW)])


_mf_kernel = functools.partial(
    pl.kernel,
    out_type=jax.ShapeDtypeStruct((B,), jnp.float32),
    mesh=plsc.VectorSubcoreMesh(core_axis_name="c", subcore_axis_name="s"),
    scratch_types=[
        pltpu.VMEM((N_CHUNKS, CHUNK), jnp.int32),
        pltpu.VMEM((N_CHUNKS, CHUNK), jnp.int32),
        pltpu.VMEM((2, D, CHUNK), jnp.int32),
        pltpu.VMEM((2, D, CHUNK), jnp.int32),
        pltpu.VMEM((2, D, CHUNK), jnp.float32),
        pltpu.VMEM((2, D, CHUNK), jnp.float32),
        pltpu.VMEM((BPW,), jnp.float32),
        pltpu.SemaphoreType.DMA((2, 2)),
    ],
    compiler_params=pltpu.CompilerParams(use_tc_tiling_on_sc=False),
)(_mf_body)


def kernel(user, anime, user_factors, anime_factors):
    ufl = jnp.transpose(user_factors).reshape(D * N_USERS)
    afl = jnp.transpose(anime_factors).reshape(D * N_ANIME)
    ufl = lax.optimization_barrier(ufl)
    afl = lax.optimization_barrier(afl)
    return _mf_kernel(user.astype(jnp.int32), anime.astype(jnp.int32),
                      ufl, afl)


# speedup vs baseline: 12.8537x; 12.8537x over previous
import functools

import jax
import jax.numpy as jnp
from jax import lax
from jax.experimental import pallas as pl
from jax.experimental.pallas import tpu as pltpu
from jax.experimental.pallas import tpu_sc as plsc

B = 16384
D = 64
NC = 2
NS = 16
NW = NC * NS
BPW = B // NW
CHUNK = 128
N_CHUNKS = BPW // CHUNK
LANES = 16
GROUPS = BPW // LANES
PIPE = 2
RING = 4
GROUP_BYTES = LANES * D * 4


def _mf_body(user_hbm, anime_hbm, uf_hbm, af_hbm, out_hbm,
             uidx, aidx, urows, arows, outv, sems):
    wid = lax.axis_index("s") * NC + lax.axis_index("c")
    base = pl.multiple_of(wid * BPW, BPW)

    for k in range(N_CHUNKS):
        pltpu.sync_copy(user_hbm.at[pl.ds(base + k * CHUNK, CHUNK)],
                        uidx.at[k])
        pltpu.sync_copy(anime_hbm.at[pl.ds(base + k * CHUNK, CHUNK)],
                        aidx.at[k])

    iota = lax.iota(jnp.int32, LANES)
    gather_dnums = lax.GatherDimensionNumbers(
        offset_dims=(), collapsed_slice_dims=(0,), start_index_map=(0,))
    rot_idx = [jnp.bitwise_and(iota + r, LANES - 1) for r in (8, 4, 2, 1)]

    def rot(x, ridx):
        return lax.gather(x, ridx[:, None], dimension_numbers=gather_dnums,
                          slice_sizes=(1,),
                          mode=lax.GatherScatterMode.PROMISE_IN_BOUNDS)

    def fire(g):
        k = g // (CHUNK // LANES)
        off = pl.multiple_of((g % (CHUNK // LANES)) * LANES, LANES)
        slot = pl.multiple_of(jnp.bitwise_and(g, RING - 1) * LANES, LANES)
        uvec = uidx[k, pl.ds(off, LANES)]
        avec = aidx[k, pl.ds(off, LANES)]
        for j in range(LANES):
            pltpu.async_copy(uf_hbm.at[pl.ds(uvec[j], 1), :],
                             urows.at[pl.ds(slot + j, 1), :], sems.at[0])
            pltpu.async_copy(af_hbm.at[pl.ds(avec[j], 1), :],
                             arows.at[pl.ds(slot + j, 1), :], sems.at[1])

    def compute(g):
        gbase = pl.multiple_of(g * LANES, LANES)
        slot = pl.multiple_of(jnp.bitwise_and(g, RING - 1) * LANES, LANES)
        outvec = jnp.zeros((LANES,), jnp.float32)
        for j in range(LANES):
            row = slot + j
            acc = (urows[row, pl.ds(0, LANES)] *
                   arows[row, pl.ds(0, LANES)])
            for c in range(1, D // LANES):
                acc = acc + (urows[row, pl.ds(c * LANES, LANES)] *
                             arows[row, pl.ds(c * LANES, LANES)])
            for ridx in rot_idx:
                acc = acc + rot(acc, ridx)
            outvec = jnp.where(iota == j, acc, outvec)
        outv[pl.ds(gbase, LANES)] = outvec

    def step(g, carry):
        @pl.when(g < GROUPS)
        def _():
            fire(g)

        @pl.when(g >= PIPE)
        def _():
            for j in range(LANES):
                pltpu.make_async_copy(uf_hbm.at[pl.ds(0, 1), :],
                                      urows.at[pl.ds(j, 1), :],
                                      sems.at[0]).wait()
                pltpu.make_async_copy(af_hbm.at[pl.ds(0, 1), :],
                                      arows.at[pl.ds(j, 1), :],
                                      sems.at[1]).wait()
            compute(g - PIPE)
        return carry

    lax.fori_loop(0, GROUPS + PIPE, step, 0)
    pltpu.sync_copy(outv, out_hbm.at[pl.ds(base, BPW)])


_mf_kernel = functools.partial(
    pl.kernel,
    out_type=jax.ShapeDtypeStruct((B,), jnp.float32),
    mesh=plsc.VectorSubcoreMesh(core_axis_name="c", subcore_axis_name="s"),
    scratch_types=[
        pltpu.VMEM((N_CHUNKS, CHUNK), jnp.int32),
        pltpu.VMEM((N_CHUNKS, CHUNK), jnp.int32),
        pltpu.VMEM((RING * LANES, D), jnp.float32),
        pltpu.VMEM((RING * LANES, D), jnp.float32),
        pltpu.VMEM((BPW,), jnp.float32),
        pltpu.SemaphoreType.DMA((2,)),
    ],
    compiler_params=pltpu.CompilerParams(use_tc_tiling_on_sc=True),
)(_mf_body)


def kernel(user, anime, user_factors, anime_factors):
    return _mf_kernel(user.astype(jnp.int32), anime.astype(jnp.int32),
                      user_factors, anime_factors)
